# baseline (speedup 1.0000x reference)
import dataclasses

import jax
import jax.numpy as jnp
from jax import lax
from jax.experimental import pallas as pl
from jax.experimental.pallas import tpu as pltpu
from jax.experimental.pallas import tpu_sc as plsc

N = 10000
E = 320000
H = 128
NUM_TASKS = 128
NUM_GRAPHS = 256
NUM_ATOM_FEATS = 9
ATOM_VOCAB = 119

NPAD = 10112
DUMMY = N
RPS = NPAD // 16

NW = 32
C = 128

A_CHUNKS = 22
A_TOTAL = NW * A_CHUNKS * C

D_CHUNKS = 80
D_TOTAL = NW * D_CHUNKS * C

EC = 64
EG = 4
NGROUPS = 42
E_CHUNKS = NGROUPS * EG
E_TOTAL = NW * E_CHUNKS * EC
E2 = E + N

_MESH = plsc.VectorSubcoreMesh(core_axis_name="c", subcore_axis_name="s")

_CP = pltpu.CompilerParams()
if "needs_layout_passes" in pltpu.CompilerParams.__dataclass_fields__:
    _CP = dataclasses.replace(_CP, needs_layout_passes=False)

LO = 0.01
HI = 1e7


def _sub_rows(ref, sub):
    return ref.at[pl.ds(sub * RPS, RPS)]


def _atom_body(emb_hbm, aidx_hbm, adst_hbm, z128_hbm, hpart_hbm,
               aidx_v, adst_v, gbuf, acc_h, gsems, ssems):
    core = lax.axis_index("c")
    sub = lax.axis_index("s")
    w = core * 16 + sub

    pltpu.sync_copy(_sub_rows(z128_hbm, sub), _sub_rows(acc_h, sub))
    plsc.subcore_barrier()

    pltpu.sync_copy(aidx_hbm.at[w], aidx_v)
    pltpu.sync_copy(adst_hbm.at[w], adst_v)

    def a_gather(j, b):
        return pltpu.make_async_copy(emb_hbm.at[aidx_v.at[j]], gbuf.at[b],
                                     gsems.at[b])

    def a_swait(j, b):
        pltpu.make_async_copy(gbuf.at[b], acc_h.at[adst_v.at[j]],
                              ssems.at[b]).wait()

    a_gather(0, 0).start()

    @pl.loop(0, A_CHUNKS, step=2)
    def _(k):
        for b in range(2):
            j = k + b
            o = 1 - b
            a_gather(j, b).wait()
            pltpu.async_copy(gbuf.at[b], acc_h.at[adst_v.at[j]],
                             ssems.at[b], add=True)

            @pl.when(j + 1 < A_CHUNKS)
            def _():
                @pl.when(j >= 1)
                def _():
                    a_swait(j - 1, o)
                a_gather(j + 1, o).start()

    a_swait(A_CHUNKS - 2, (A_CHUNKS - 2) % 2)
    a_swait(A_CHUNKS - 1, (A_CHUNKS - 1) % 2)

    plsc.subcore_barrier()
    pltpu.sync_copy(_sub_rows(acc_h, sub),
                    hpart_hbm.at[core].at[pl.ds(sub * RPS, RPS)])


def _atom(emb_flat, aidx, adst, z128):
    kern = pl.kernel(
        _atom_body,
        out_type=jax.ShapeDtypeStruct((2, NPAD, H), jnp.float32),
        mesh=_MESH,
        scratch_types=[
            pltpu.VMEM((A_CHUNKS, C), jnp.int32),
            pltpu.VMEM((A_CHUNKS, C), jnp.int32),
            pltpu.VMEM((2, C, H), jnp.float32),
            pltpu.VMEM_SHARED((NPAD, H), jnp.float32),
            pltpu.SemaphoreType.DMA((2,)),
            pltpu.SemaphoreType.DMA((2,)),
        ],
    )
    return kern(emb_flat, aidx, adst, z128)


def _deg_body(val_hbm, cidx_hbm, z16_hbm, dpart_hbm,
              cidx_v, vbuf, acc_d, gsems, ssems):
    core = lax.axis_index("c")
    sub = lax.axis_index("s")
    w = core * 16 + sub

    pltpu.sync_copy(_sub_rows(z16_hbm, sub), _sub_rows(acc_d, sub))
    plsc.subcore_barrier()

    pltpu.sync_copy(cidx_hbm.at[w], cidx_v)

    def d_load(j, b):
        return pltpu.make_async_copy(val_hbm.at[w].at[j], vbuf.at[b],
                                     gsems.at[b])

    def d_swait(j, b):
        pltpu.make_async_copy(vbuf.at[b], acc_d.at[cidx_v.at[j]],
                              ssems.at[b]).wait()

    d_load(0, 0).start()

    @pl.loop(0, D_CHUNKS, step=2)
    def _(k):
        for b in range(2):
            j = k + b
            o = 1 - b
            d_load(j, b).wait()
            pltpu.async_copy(vbuf.at[b], acc_d.at[cidx_v.at[j]],
                             ssems.at[b], add=True)

            @pl.when(j + 1 < D_CHUNKS)
            def _():
                @pl.when(j >= 1)
                def _():
                    d_swait(j - 1, o)
                d_load(j + 1, o).start()

    d_swait(D_CHUNKS - 2, (D_CHUNKS - 2) % 2)
    d_swait(D_CHUNKS - 1, (D_CHUNKS - 1) % 2)

    plsc.subcore_barrier()
    pltpu.sync_copy(_sub_rows(acc_d, sub),
                    dpart_hbm.at[core].at[pl.ds(sub * RPS, RPS)])


def _deg(val16, cidx_d, z16):
    kern = pl.kernel(
        _deg_body,
        out_type=jax.ShapeDtypeStruct((2, NPAD, 16), jnp.float32),
        mesh=_MESH,
        scratch_types=[
            pltpu.VMEM((D_CHUNKS, C), jnp.int32),
            pltpu.VMEM((2, C, 16), jnp.float32),
            pltpu.VMEM_SHARED((NPAD, 16), jnp.float32),
            pltpu.SemaphoreType.DMA((2,)),
            pltpu.SemaphoreType.DMA((2,)),
        ],
    )
    return kern(val16, cidx_d, z16)


def _invn_body(sq_hbm, row_hbm, col_hbm, ew_hbm, inv_hbm,
               sq_v, row_v, col_v, ew_v, inv_v):
    core = lax.axis_index("c")
    sub = lax.axis_index("s")
    w = core * 16 + sub

    pltpu.sync_copy(sq_hbm, sq_v)
    pltpu.sync_copy(row_hbm.at[w], row_v)
    pltpu.sync_copy(col_hbm.at[w], col_v)
    pltpu.sync_copy(ew_hbm.at[w], ew_v)

    @pl.loop(0, E_CHUNKS, step=2)
    def _(j0):
        for dj in range(2):
            j = j0 + dj
            for t in range(EC // 16):
                o = t * 16
                r16 = row_v[j, pl.ds(o, 16)]
                c16 = col_v[j, pl.ds(o, 16)]
                e16 = ew_v[j, pl.ds(o, 16)]
                v = (plsc.load_gather(sq_v, [r16])
                     * plsc.load_gather(sq_v, [c16]))
                inv_v[j, pl.ds(o, 16)] = v / e16

    pltpu.sync_copy(inv_v, inv_hbm.at[w])


def _invn(sq, rowp, colp, ewp):
    kern = pl.kernel(
        _invn_body,
        out_type=jax.ShapeDtypeStruct((NW, E_CHUNKS, EC), jnp.float32),
        mesh=_MESH,
        scratch_types=[
            pltpu.VMEM((NPAD,), jnp.float32),
            pltpu.VMEM((E_CHUNKS, EC), jnp.int32),
            pltpu.VMEM((E_CHUNKS, EC), jnp.int32),
            pltpu.VMEM((E_CHUNKS, EC), jnp.float32),
            pltpu.VMEM((E_CHUNKS, EC), jnp.float32),
        ],
        compiler_params=_CP,
    )
    return kern(sq, rowp, colp, ewp)


def _edge_body(u_hbm, row_hbm, col_hbm, inv_hbm, z128_hbm, spart_hbm,
               rbuf, cbuf, ibuf, gbuf, sbuf, acc, gsems, ssems):
    core = lax.axis_index("c")
    sub = lax.axis_index("s")
    w = core * 16 + sub

    pltpu.sync_copy(_sub_rows(z128_hbm, sub), _sub_rows(acc, sub))
    plsc.subcore_barrier()

    def stage(g, p):
        pltpu.sync_copy(row_hbm.at[w].at[pl.ds(g * EG, EG)], rbuf.at[p])
        pltpu.sync_copy(col_hbm.at[w].at[pl.ds(g * EG, EG)], cbuf.at[p])
        pltpu.sync_copy(inv_hbm.at[w].at[pl.ds(g * EG, EG)], ibuf.at[p])

    def g_start(p, cc, b):
        pltpu.make_async_copy(u_hbm.at[rbuf.at[p].at[cc]], gbuf.at[b],
                              gsems.at[b]).start()

    def g_wait(p, cc, b):
        pltpu.make_async_copy(u_hbm.at[rbuf.at[p].at[cc]], gbuf.at[b],
                              gsems.at[b]).wait()

    def s_wait(p, cc, b):
        pltpu.make_async_copy(sbuf.at[b], acc.at[cbuf.at[p].at[cc]],
                              ssems.at[b]).wait()

    stage(0, 0)
    g_start(0, 0, 0)
    g_start(0, 1, 1)

    @pl.loop(0, NGROUPS, step=2)
    def _(gg):
        for p in range(2):
            g = gg + p
            for cc in range(4):
                j4 = g * 4 + cc
                b = cc % 2
                g_wait(p, cc, b)

                @pl.when(j4 >= 2)
                def _():
                    if cc < 2:
                        s_wait(1 - p, cc + 2, b)
                    else:
                        s_wait(p, cc - 2, b)

                gb = gbuf.at[b]
                sb = sbuf.at[b]
                iv = ibuf.at[p].at[cc]

                @pl.loop(0, EC, step=4)
                def _(e0):
                    for de in range(4):
                        e = e0 + de
                        inv16 = plsc.load_gather(
                            iv, [jnp.full((16,), e, jnp.int32)])
                        for t in range(8):
                            sl = pl.ds(t * 16, 16)
                            v = gb[e, sl] * inv16
                            sb[e, sl] = jnp.minimum(jnp.maximum(v, LO), HI)

                pltpu.async_copy(sbuf.at[b], acc.at[cbuf.at[p].at[cc]],
                                 ssems.at[b], add=True)

                if cc < 2:
                    g_start(p, cc + 2, b)
                    if cc == 1:
                        @pl.when(g + 1 < NGROUPS)
                        def _():
                            stage(g + 1, 1 - p)
                else:
                    @pl.when(g + 1 < NGROUPS)
                    def _():
                        g_start(1 - p, cc - 2, b)

    s_wait(1, 2, 0)
    s_wait(1, 3, 1)

    plsc.subcore_barrier()
    pltpu.sync_copy(_sub_rows(acc, sub),
                    spart_hbm.at[core].at[pl.ds(sub * RPS, RPS)])


def _edge_pass(u, rowp, colp, invp, z128):
    kern = pl.kernel(
        _edge_body,
        out_type=jax.ShapeDtypeStruct((2, NPAD, H), jnp.float32),
        mesh=_MESH,
        scratch_types=[
            pltpu.VMEM((2, EG, EC), jnp.int32),
            pltpu.VMEM((2, EG, EC), jnp.int32),
            pltpu.VMEM((2, EG, EC), jnp.float32),
            pltpu.VMEM((2, EC, H), jnp.float32),
            pltpu.VMEM((2, EC, H), jnp.float32),
            pltpu.VMEM_SHARED((NPAD, H), jnp.float32),
            pltpu.SemaphoreType.DMA((2,)),
            pltpu.SemaphoreType.DMA((2,)),
        ],
        compiler_params=_CP,
    )
    return kern(u, rowp, colp, invp, z128)


def _sq_body(dpart, sq_out, cnt_out):
    deg = dpart[0, :, 0:1] + dpart[1, :, 0:1] + 1.0
    cnt = dpart[0, :N, 1:2] + dpart[1, :N, 1:2] + 1.0
    sq_out[...] = jnp.sqrt(deg)
    cnt_out[...] = cnt


def _mm1_body(hpart, w1, u_out):
    h = hpart[0, :N, :] + hpart[1, :N, :]
    hw = jnp.dot(h, w1[...], preferred_element_type=jnp.float32)
    u_out[...] = jnp.where(hw > 0, 1.0 / hw, jnp.inf)


def _mm2_body(spart, cnt, b1, w2, u_out):
    s = spart[0, :N, :] + spart[1, :N, :]
    pre = jnp.clip(cnt[...] / s, LO, HI) + b1[...][None, :]
    h1 = jnp.maximum(pre, 0.0)
    hw = jnp.dot(h1, w2[...], preferred_element_type=jnp.float32)
    u_out[...] = jnp.where(hw > 0, 1.0 / hw, jnp.inf)


def _final_body(spart, cnt, b2, batch, lin_w, lin_b, out):
    s = spart[0, :N, :] + spart[1, :N, :]
    h2 = jnp.clip(cnt[...] / s, LO, HI) + b2[...][None, :]
    iota = lax.broadcasted_iota(jnp.int32, (N, NUM_GRAPHS), 1)
    oh = (batch[...] == iota).astype(jnp.float32)
    gsum = lax.dot_general(oh, h2, (((0,), (0,)), ((), ())),
                           preferred_element_type=jnp.float32)
    gcnt = jnp.sum(oh, axis=0)
    g = gsum / jnp.maximum(gcnt, 1.0)[:, None]
    out[...] = jnp.dot(g, lin_w[...],
                       preferred_element_type=jnp.float32) + lin_b[...][None, :]


def _tc_call(body, out_shape, *args):
    return pl.pallas_call(body, out_shape=out_shape)(*args)


@jax.jit
def kernel(x, edge_index, batch, edge_weight, atom_emb, W1, b1, W2, b2,
           lin_W, lin_b):
    f32, i32 = jnp.float32, jnp.int32
    row, col = edge_index[0], edge_index[1]

    emb_flat = atom_emb.reshape(NUM_ATOM_FEATS * ATOM_VOCAB, H)
    offs = (jnp.arange(NUM_ATOM_FEATS, dtype=i32) * ATOM_VOCAB)[None, :]
    aflat = (x + offs).reshape(-1)
    aflat = jnp.concatenate(
        [aflat, jnp.zeros((A_TOTAL - N * NUM_ATOM_FEATS,), i32)])
    aidx = aflat.reshape(NW, A_CHUNKS, C)
    adst = jnp.repeat(jnp.arange(N, dtype=i32), NUM_ATOM_FEATS)
    adst = jnp.concatenate(
        [adst, jnp.full((A_TOTAL - N * NUM_ATOM_FEATS,), DUMMY, i32)])
    adst = adst.reshape(NW, A_CHUNKS, C)

    val16 = jnp.concatenate(
        [edge_weight[:, None], jnp.ones((E, 1), f32), jnp.zeros((E, 14), f32)],
        axis=1)
    val16 = jnp.concatenate([val16, jnp.zeros((D_TOTAL - E, 16), f32)])
    val16 = val16.reshape(NW, D_CHUNKS, C, 16)
    cidx_d = jnp.concatenate([col, jnp.full((D_TOTAL - E,), DUMMY, i32)])
    cidx_d = cidx_d.reshape(NW, D_CHUNKS, C)

    loop = jnp.arange(N, dtype=i32)
    npadE = E_TOTAL - E2
    rowp = jnp.concatenate([row, loop, jnp.zeros((npadE,), i32)])
    colp = jnp.concatenate([col, loop, jnp.full((npadE,), DUMMY, i32)])
    ewp = jnp.concatenate([edge_weight, jnp.ones((N + npadE,), f32)])
    rowp = rowp.reshape(NW, E_CHUNKS, EC)
    colp = colp.reshape(NW, E_CHUNKS, EC)
    ewp = ewp.reshape(NW, E_CHUNKS, EC)

    z128 = jnp.zeros((NPAD, H), f32)
    z16 = jnp.zeros((NPAD, 16), f32)

    hpart = _atom(emb_flat, aidx, adst, z128)
    dpart = _deg(val16, cidx_d, z16)

    sq, cnt = _tc_call(
        _sq_body,
        [jax.ShapeDtypeStruct((NPAD, 1), f32),
         jax.ShapeDtypeStruct((N, 1), f32)],
        dpart)
    sq1d = sq.reshape(NPAD)

    invp = _invn(sq1d, rowp, colp, ewp)
    u1 = _tc_call(_mm1_body, jax.ShapeDtypeStruct((N, H), f32), hpart, W1)
    spart1 = _edge_pass(u1, rowp, colp, invp, z128)

    u2 = _tc_call(_mm2_body, jax.ShapeDtypeStruct((N, H), f32),
                  spart1, cnt, b1, W2)

    spart2 = _edge_pass(u2, rowp, colp, invp, z128)

    out = _tc_call(_final_body,
                   jax.ShapeDtypeStruct((NUM_GRAPHS, NUM_TASKS), f32),
                   spart2, cnt, b2, batch[:, None], lin_W, lin_b)
    return out


# speedup vs baseline: 1.0001x; 1.0001x over previous
import dataclasses

import jax
import jax.numpy as jnp
from jax import lax
from jax.experimental import pallas as pl
from jax.experimental.pallas import tpu as pltpu
from jax.experimental.pallas import tpu_sc as plsc

N = 10000
E = 320000
H = 128
NUM_TASKS = 128
NUM_GRAPHS = 256
NUM_ATOM_FEATS = 9
ATOM_VOCAB = 119

NPAD = 10112
DUMMY = N
RPS = NPAD // 16

NW = 32
C = 128

A_CHUNKS = 22
A_TOTAL = NW * A_CHUNKS * C

D_CHUNKS = 80
D_TOTAL = NW * D_CHUNKS * C

EC = 64
EG = 4
NGROUPS = 42
E_CHUNKS = NGROUPS * EG
E_TOTAL = NW * E_CHUNKS * EC
E2 = E + N

_MESH = plsc.VectorSubcoreMesh(core_axis_name="c", subcore_axis_name="s")

_CP = pltpu.CompilerParams()
if "needs_layout_passes" in pltpu.CompilerParams.__dataclass_fields__:
    _CP = dataclasses.replace(_CP, needs_layout_passes=False)

LO = 0.01
HI = 1e7


def _sub_rows(ref, sub):
    return ref.at[pl.ds(sub * RPS, RPS)]


def _atom_body(emb_hbm, aidx_hbm, adst_hbm, z128_hbm, hpart_hbm,
               aidx_v, adst_v, gbuf, acc_h, gsems, ssems):
    core = lax.axis_index("c")
    sub = lax.axis_index("s")
    w = core * 16 + sub

    pltpu.sync_copy(_sub_rows(z128_hbm, sub), _sub_rows(acc_h, sub))
    plsc.subcore_barrier()

    pltpu.sync_copy(aidx_hbm.at[w], aidx_v)
    pltpu.sync_copy(adst_hbm.at[w], adst_v)

    def a_gather(j, b):
        return pltpu.make_async_copy(emb_hbm.at[aidx_v.at[j]], gbuf.at[b],
                                     gsems.at[b])

    def a_swait(j, b):
        pltpu.make_async_copy(gbuf.at[b], acc_h.at[adst_v.at[j]],
                              ssems.at[b]).wait()

    a_gather(0, 0).start()

    @pl.loop(0, A_CHUNKS, step=2)
    def _(k):
        for b in range(2):
            j = k + b
            o = 1 - b
            a_gather(j, b).wait()
            pltpu.async_copy(gbuf.at[b], acc_h.at[adst_v.at[j]],
                             ssems.at[b], add=True)

            @pl.when(j + 1 < A_CHUNKS)
            def _():
                @pl.when(j >= 1)
                def _():
                    a_swait(j - 1, o)
                a_gather(j + 1, o).start()

    a_swait(A_CHUNKS - 2, (A_CHUNKS - 2) % 2)
    a_swait(A_CHUNKS - 1, (A_CHUNKS - 1) % 2)

    plsc.subcore_barrier()
    pltpu.sync_copy(_sub_rows(acc_h, sub),
                    hpart_hbm.at[core].at[pl.ds(sub * RPS, RPS)])


def _atom(emb_flat, aidx, adst, z128):
    kern = pl.kernel(
        _atom_body,
        out_type=jax.ShapeDtypeStruct((2, NPAD, H), jnp.float32),
        mesh=_MESH,
        scratch_types=[
            pltpu.VMEM((A_CHUNKS, C), jnp.int32),
            pltpu.VMEM((A_CHUNKS, C), jnp.int32),
            pltpu.VMEM((2, C, H), jnp.float32),
            pltpu.VMEM_SHARED((NPAD, H), jnp.float32),
            pltpu.SemaphoreType.DMA((2,)),
            pltpu.SemaphoreType.DMA((2,)),
        ],
    )
    return kern(emb_flat, aidx, adst, z128)


def _deg_body(val_hbm, cidx_hbm, z16_hbm, dpart_hbm,
              cidx_v, vbuf, acc_d, gsems, ssems):
    core = lax.axis_index("c")
    sub = lax.axis_index("s")
    w = core * 16 + sub

    pltpu.sync_copy(_sub_rows(z16_hbm, sub), _sub_rows(acc_d, sub))
    plsc.subcore_barrier()

    pltpu.sync_copy(cidx_hbm.at[w], cidx_v)

    def d_load(j, b):
        return pltpu.make_async_copy(val_hbm.at[w].at[j], vbuf.at[b],
                                     gsems.at[b])

    def d_swait(j, b):
        pltpu.make_async_copy(vbuf.at[b], acc_d.at[cidx_v.at[j]],
                              ssems.at[b]).wait()

    d_load(0, 0).start()

    @pl.loop(0, D_CHUNKS, step=2)
    def _(k):
        for b in range(2):
            j = k + b
            o = 1 - b
            d_load(j, b).wait()
            pltpu.async_copy(vbuf.at[b], acc_d.at[cidx_v.at[j]],
                             ssems.at[b], add=True)

            @pl.when(j + 1 < D_CHUNKS)
            def _():
                @pl.when(j >= 1)
                def _():
                    d_swait(j - 1, o)
                d_load(j + 1, o).start()

    d_swait(D_CHUNKS - 2, (D_CHUNKS - 2) % 2)
    d_swait(D_CHUNKS - 1, (D_CHUNKS - 1) % 2)

    plsc.subcore_barrier()
    pltpu.sync_copy(_sub_rows(acc_d, sub),
                    dpart_hbm.at[core].at[pl.ds(sub * RPS, RPS)])


def _deg(val16, cidx_d, z16):
    kern = pl.kernel(
        _deg_body,
        out_type=jax.ShapeDtypeStruct((2, NPAD, 16), jnp.float32),
        mesh=_MESH,
        scratch_types=[
            pltpu.VMEM((D_CHUNKS, C), jnp.int32),
            pltpu.VMEM((2, C, 16), jnp.float32),
            pltpu.VMEM_SHARED((NPAD, 16), jnp.float32),
            pltpu.SemaphoreType.DMA((2,)),
            pltpu.SemaphoreType.DMA((2,)),
        ],
    )
    return kern(val16, cidx_d, z16)


def _invn_body(sq_hbm, row_hbm, col_hbm, ew_hbm, inv_hbm,
               sq_v, row_v, col_v, ew_v, inv_v):
    core = lax.axis_index("c")
    sub = lax.axis_index("s")
    w = core * 16 + sub

    pltpu.sync_copy(sq_hbm, sq_v)
    pltpu.sync_copy(row_hbm.at[w], row_v)
    pltpu.sync_copy(col_hbm.at[w], col_v)
    pltpu.sync_copy(ew_hbm.at[w], ew_v)

    @pl.loop(0, E_CHUNKS, step=2)
    def _(j0):
        for dj in range(2):
            j = j0 + dj
            for t in range(EC // 16):
                o = t * 16
                r16 = row_v[j, pl.ds(o, 16)]
                c16 = col_v[j, pl.ds(o, 16)]
                e16 = ew_v[j, pl.ds(o, 16)]
                v = (plsc.load_gather(sq_v, [r16])
                     * plsc.load_gather(sq_v, [c16]))
                inv_v[j, pl.ds(o, 16)] = v / e16

    pltpu.sync_copy(inv_v, inv_hbm.at[w])


def _invn(sq, rowp, colp, ewp):
    kern = pl.kernel(
        _invn_body,
        out_type=jax.ShapeDtypeStruct((NW, E_CHUNKS, EC), jnp.float32),
        mesh=_MESH,
        scratch_types=[
            pltpu.VMEM((NPAD,), jnp.float32),
            pltpu.VMEM((E_CHUNKS, EC), jnp.int32),
            pltpu.VMEM((E_CHUNKS, EC), jnp.int32),
            pltpu.VMEM((E_CHUNKS, EC), jnp.float32),
            pltpu.VMEM((E_CHUNKS, EC), jnp.float32),
        ],
        compiler_params=_CP,
    )
    return kern(sq, rowp, colp, ewp)


def _edge_body(u_hbm, row_hbm, col_hbm, inv_hbm, z128_hbm, spart_hbm,
               rbuf, cbuf, ibuf, gbuf, sbuf, acc, gsems, ssems):
    core = lax.axis_index("c")
    sub = lax.axis_index("s")
    w = core * 16 + sub

    pltpu.sync_copy(_sub_rows(z128_hbm, sub), _sub_rows(acc, sub))
    plsc.subcore_barrier()

    def stage(g, p):
        pltpu.sync_copy(row_hbm.at[w].at[pl.ds(g * EG, EG)], rbuf.at[p])
        pltpu.sync_copy(col_hbm.at[w].at[pl.ds(g * EG, EG)], cbuf.at[p])
        pltpu.sync_copy(inv_hbm.at[w].at[pl.ds(g * EG, EG)], ibuf.at[p])

    def g_start(p, cc, b):
        pltpu.make_async_copy(u_hbm.at[rbuf.at[p].at[cc]], gbuf.at[b],
                              gsems.at[b]).start()

    def g_wait(p, cc, b):
        pltpu.make_async_copy(u_hbm.at[rbuf.at[p].at[cc]], gbuf.at[b],
                              gsems.at[b]).wait()

    def s_wait(p, cc, b):
        pltpu.make_async_copy(sbuf.at[b], acc.at[cbuf.at[p].at[cc]],
                              ssems.at[b]).wait()

    stage(0, 0)
    g_start(0, 0, 0)
    g_start(0, 1, 1)

    @pl.loop(0, NGROUPS, step=2)
    def _(gg):
        for p in range(2):
            g = gg + p
            for cc in range(4):
                j4 = g * 4 + cc
                b = cc % 2
                g_wait(p, cc, b)

                @pl.when(j4 >= 2)
                def _():
                    if cc < 2:
                        s_wait(1 - p, cc + 2, b)
                    else:
                        s_wait(p, cc - 2, b)

                gb = gbuf.at[b]
                sb = sbuf.at[b]
                iv = ibuf.at[p].at[cc]

                @pl.loop(0, EC, step=4)
                def _(e0):
                    for de in range(4):
                        e = e0 + de
                        inv16 = plsc.load_gather(
                            iv, [jnp.full((16,), e, jnp.int32)])
                        for t in range(8):
                            sl = pl.ds(t * 16, 16)
                            v = gb[e, sl] * inv16
                            sb[e, sl] = jnp.minimum(jnp.maximum(v, LO), HI)

                pltpu.async_copy(sbuf.at[b], acc.at[cbuf.at[p].at[cc]],
                                 ssems.at[b], add=True)

                if cc < 2:
                    g_start(p, cc + 2, b)
                    if cc == 1:
                        @pl.when(g + 1 < NGROUPS)
                        def _():
                            stage(g + 1, 1 - p)
                else:
                    @pl.when(g + 1 < NGROUPS)
                    def _():
                        g_start(1 - p, cc - 2, b)

    s_wait(1, 2, 0)
    s_wait(1, 3, 1)

    plsc.subcore_barrier()
    pltpu.sync_copy(_sub_rows(acc, sub),
                    spart_hbm.at[core].at[pl.ds(sub * RPS, RPS)])


def _edge_pass(u, rowp, colp, invp, z128):
    kern = pl.kernel(
        _edge_body,
        out_type=jax.ShapeDtypeStruct((2, NPAD, H), jnp.float32),
        mesh=_MESH,
        scratch_types=[
            pltpu.VMEM((2, EG, EC), jnp.int32),
            pltpu.VMEM((2, EG, EC), jnp.int32),
            pltpu.VMEM((2, EG, EC), jnp.float32),
            pltpu.VMEM((2, EC, H), jnp.float32),
            pltpu.VMEM((2, EC, H), jnp.float32),
            pltpu.VMEM_SHARED((NPAD, H), jnp.float32),
            pltpu.SemaphoreType.DMA((2,)),
            pltpu.SemaphoreType.DMA((2,)),
        ],
        compiler_params=_CP,
    )
    return kern(u, rowp, colp, invp, z128)


def _sq_body(dpart, sq_out, cnt_out):
    deg = dpart[0, :, 0:1] + dpart[1, :, 0:1] + 1.0
    cnt = dpart[0, :N, 1:2] + dpart[1, :N, 1:2] + 1.0
    sq_out[...] = jnp.sqrt(deg)
    cnt_out[...] = cnt


def _mm1_body(hpart, w1, u_out):
    h = hpart[0, :N, :] + hpart[1, :N, :]
    hw = jnp.dot(h, w1[...], preferred_element_type=jnp.float32)
    u_out[...] = jnp.where(hw > 0, 1.0 / hw, jnp.inf)


def _mm2_body(spart, cnt, b1, w2, u_out):
    s = spart[0, :N, :] + spart[1, :N, :]
    pre = jnp.clip(cnt[...] / s, LO, HI) + b1[...][None, :]
    h1 = jnp.maximum(pre, 0.0)
    hw = jnp.dot(h1, w2[...], preferred_element_type=jnp.float32)
    u_out[...] = jnp.where(hw > 0, 1.0 / hw, jnp.inf)


def _final_body(spart, cnt, b2, batch, lin_w, lin_b, out):
    s = spart[0, :N, :] + spart[1, :N, :]
    h2 = jnp.clip(cnt[...] / s, LO, HI) + b2[...][None, :]
    iota = lax.broadcasted_iota(jnp.int32, (N, NUM_GRAPHS), 1)
    oh = (batch[...] == iota).astype(jnp.float32)
    gsum = lax.dot_general(oh, h2, (((0,), (0,)), ((), ())),
                           preferred_element_type=jnp.float32)
    gcnt = jnp.sum(oh, axis=0)
    g = gsum / jnp.maximum(gcnt, 1.0)[:, None]
    out[...] = jnp.dot(g, lin_w[...],
                       preferred_element_type=jnp.float32) + lin_b[...][None, :]


def _tc_call(body, out_shape, *args):
    return pl.pallas_call(body, out_shape=out_shape)(*args)


@jax.jit
def kernel(x, edge_index, batch, edge_weight, atom_emb, W1, b1, W2, b2,
           lin_W, lin_b):
    f32, i32 = jnp.float32, jnp.int32
    row, col = edge_index[0], edge_index[1]

    emb_flat = atom_emb.reshape(NUM_ATOM_FEATS * ATOM_VOCAB, H)
    offs = (jnp.arange(NUM_ATOM_FEATS, dtype=i32) * ATOM_VOCAB)[None, :]
    aflat = (x + offs).reshape(-1)
    aflat = jnp.concatenate(
        [aflat, jnp.zeros((A_TOTAL - N * NUM_ATOM_FEATS,), i32)])
    aidx = aflat.reshape(NW, A_CHUNKS, C)
    adst = jnp.repeat(jnp.arange(N, dtype=i32), NUM_ATOM_FEATS)
    apad = A_TOTAL - N * NUM_ATOM_FEATS
    adst = jnp.concatenate(
        [adst, DUMMY + (jnp.arange(apad, dtype=i32) % (NPAD - N))])
    adst = adst.reshape(NW, A_CHUNKS, C)

    val16 = jnp.concatenate(
        [edge_weight[:, None], jnp.ones((E, 1), f32), jnp.zeros((E, 14), f32)],
        axis=1)
    val16 = jnp.concatenate([val16, jnp.zeros((D_TOTAL - E, 16), f32)])
    val16 = val16.reshape(NW, D_CHUNKS, C, 16)
    cidx_d = jnp.concatenate(
        [col, DUMMY + (jnp.arange(D_TOTAL - E, dtype=i32) % (NPAD - N))])
    cidx_d = cidx_d.reshape(NW, D_CHUNKS, C)

    loop = jnp.arange(N, dtype=i32)
    npadE = E_TOTAL - E2
    pad_col = DUMMY + (jnp.arange(npadE, dtype=i32) % (NPAD - N))
    rowp = jnp.concatenate([row, loop, jnp.zeros((npadE,), i32)])
    colp = jnp.concatenate([col, loop, pad_col])
    ewp = jnp.concatenate([edge_weight, jnp.ones((N + npadE,), f32)])
    rowp = rowp.reshape(NW, E_CHUNKS, EC)
    colp = colp.reshape(NW, E_CHUNKS, EC)
    ewp = ewp.reshape(NW, E_CHUNKS, EC)

    z128 = jnp.zeros((NPAD, H), f32)
    z16 = jnp.zeros((NPAD, 16), f32)

    hpart = _atom(emb_flat, aidx, adst, z128)
    dpart = _deg(val16, cidx_d, z16)

    sq, cnt = _tc_call(
        _sq_body,
        [jax.ShapeDtypeStruct((NPAD, 1), f32),
         jax.ShapeDtypeStruct((N, 1), f32)],
        dpart)
    sq1d = sq.reshape(NPAD)

    invp = _invn(sq1d, rowp, colp, ewp)
    u1 = _tc_call(_mm1_body, jax.ShapeDtypeStruct((N, H), f32), hpart, W1)
    spart1 = _edge_pass(u1, rowp, colp, invp, z128)

    u2 = _tc_call(_mm2_body, jax.ShapeDtypeStruct((N, H), f32),
                  spart1, cnt, b1, W2)

    spart2 = _edge_pass(u2, rowp, colp, invp, z128)

    out = _tc_call(_final_body,
                   jax.ShapeDtypeStruct((NUM_GRAPHS, NUM_TASKS), f32),
                   spart2, cnt, b2, batch[:, None], lin_W, lin_b)
    return out


# speedup vs baseline: 1.0181x; 1.0180x over previous
import dataclasses

import jax
import jax.numpy as jnp
from jax import lax
from jax.experimental import pallas as pl
from jax.experimental.pallas import tpu as pltpu
from jax.experimental.pallas import tpu_sc as plsc

N = 10000
E = 320000
H = 128
NUM_TASKS = 128
NUM_GRAPHS = 256
NUM_ATOM_FEATS = 9
ATOM_VOCAB = 119

NPAD = 10112
DUMMY = N
RPS = NPAD // 16

NW = 32
C = 128

A_CHUNKS = 22
A_TOTAL = NW * A_CHUNKS * C

D_CHUNKS = 80
D_TOTAL = NW * D_CHUNKS * C

EC = 64
EG = 4
NGROUPS = 42
E_CHUNKS = NGROUPS * EG
E_TOTAL = NW * E_CHUNKS * EC
E2 = E + N

_MESH = plsc.VectorSubcoreMesh(core_axis_name="c", subcore_axis_name="s")

_CP = pltpu.CompilerParams()
if "needs_layout_passes" in pltpu.CompilerParams.__dataclass_fields__:
    _CP = dataclasses.replace(_CP, needs_layout_passes=False)

LO = 0.01
HI = 1e7


def _sub_rows(ref, sub):
    return ref.at[pl.ds(sub * RPS, RPS)]


def _atom_body(emb_hbm, aidx_hbm, adst_hbm, z128_hbm, hpart_hbm,
               aidx_v, adst_v, gbuf, acc_h, gsems, ssems):
    core = lax.axis_index("c")
    sub = lax.axis_index("s")
    w = core * 16 + sub

    pltpu.sync_copy(_sub_rows(z128_hbm, sub), _sub_rows(acc_h, sub))
    plsc.subcore_barrier()

    pltpu.sync_copy(aidx_hbm.at[w], aidx_v)
    pltpu.sync_copy(adst_hbm.at[w], adst_v)

    def a_gather(j, b):
        return pltpu.make_async_copy(emb_hbm.at[aidx_v.at[j]], gbuf.at[b],
                                     gsems.at[b])

    def a_swait(j, b):
        pltpu.make_async_copy(gbuf.at[b], acc_h.at[adst_v.at[j]],
                              ssems.at[b]).wait()

    a_gather(0, 0).start()

    @pl.loop(0, A_CHUNKS, step=2)
    def _(k):
        for b in range(2):
            j = k + b
            o = 1 - b
            a_gather(j, b).wait()
            pltpu.async_copy(gbuf.at[b], acc_h.at[adst_v.at[j]],
                             ssems.at[b], add=True)

            @pl.when(j + 1 < A_CHUNKS)
            def _():
                @pl.when(j >= 1)
                def _():
                    a_swait(j - 1, o)
                a_gather(j + 1, o).start()

    a_swait(A_CHUNKS - 2, (A_CHUNKS - 2) % 2)
    a_swait(A_CHUNKS - 1, (A_CHUNKS - 1) % 2)

    plsc.subcore_barrier()
    pltpu.sync_copy(_sub_rows(acc_h, sub),
                    hpart_hbm.at[core].at[pl.ds(sub * RPS, RPS)])


def _atom(emb_flat, aidx, adst, z128):
    kern = pl.kernel(
        _atom_body,
        out_type=jax.ShapeDtypeStruct((2, NPAD, H), jnp.float32),
        mesh=_MESH,
        scratch_types=[
            pltpu.VMEM((A_CHUNKS, C), jnp.int32),
            pltpu.VMEM((A_CHUNKS, C), jnp.int32),
            pltpu.VMEM((2, C, H), jnp.float32),
            pltpu.VMEM_SHARED((NPAD, H), jnp.float32),
            pltpu.SemaphoreType.DMA((2,)),
            pltpu.SemaphoreType.DMA((2,)),
        ],
    )
    return kern(emb_flat, aidx, adst, z128)


def _deg_body(val_hbm, cidx_hbm, z16_hbm, dpart_hbm,
              cidx_v, vbuf, acc_d, gsems, ssems):
    core = lax.axis_index("c")
    sub = lax.axis_index("s")
    w = core * 16 + sub

    pltpu.sync_copy(_sub_rows(z16_hbm, sub), _sub_rows(acc_d, sub))
    plsc.subcore_barrier()

    pltpu.sync_copy(cidx_hbm.at[w], cidx_v)

    def d_load(j, b):
        return pltpu.make_async_copy(val_hbm.at[w].at[j], vbuf.at[b],
                                     gsems.at[b])

    def d_swait(j, b):
        pltpu.make_async_copy(vbuf.at[b], acc_d.at[cidx_v.at[j]],
                              ssems.at[b]).wait()

    d_load(0, 0).start()

    @pl.loop(0, D_CHUNKS, step=2)
    def _(k):
        for b in range(2):
            j = k + b
            o = 1 - b
            d_load(j, b).wait()
            pltpu.async_copy(vbuf.at[b], acc_d.at[cidx_v.at[j]],
                             ssems.at[b], add=True)

            @pl.when(j + 1 < D_CHUNKS)
            def _():
                @pl.when(j >= 1)
                def _():
                    d_swait(j - 1, o)
                d_load(j + 1, o).start()

    d_swait(D_CHUNKS - 2, (D_CHUNKS - 2) % 2)
    d_swait(D_CHUNKS - 1, (D_CHUNKS - 1) % 2)

    plsc.subcore_barrier()
    pltpu.sync_copy(_sub_rows(acc_d, sub),
                    dpart_hbm.at[core].at[pl.ds(sub * RPS, RPS)])


def _deg(val16, cidx_d, z16):
    kern = pl.kernel(
        _deg_body,
        out_type=jax.ShapeDtypeStruct((2, NPAD, 16), jnp.float32),
        mesh=_MESH,
        scratch_types=[
            pltpu.VMEM((D_CHUNKS, C), jnp.int32),
            pltpu.VMEM((2, C, 16), jnp.float32),
            pltpu.VMEM_SHARED((NPAD, 16), jnp.float32),
            pltpu.SemaphoreType.DMA((2,)),
            pltpu.SemaphoreType.DMA((2,)),
        ],
    )
    return kern(val16, cidx_d, z16)


def _invn_body(sq_hbm, row_hbm, col_hbm, ew_hbm, inv_hbm,
               sq_v, row_v, col_v, ew_v, inv_v):
    core = lax.axis_index("c")
    sub = lax.axis_index("s")
    w = core * 16 + sub

    pltpu.sync_copy(sq_hbm, sq_v)
    pltpu.sync_copy(row_hbm.at[w], row_v)
    pltpu.sync_copy(col_hbm.at[w], col_v)
    pltpu.sync_copy(ew_hbm.at[w], ew_v)

    @plsc.parallel_loop(0, E_CHUNKS, unroll=2)
    def _(j):
        for t in range(EC // 16):
            o = t * 16
            r16 = row_v[j, pl.ds(o, 16)]
            c16 = col_v[j, pl.ds(o, 16)]
            e16 = ew_v[j, pl.ds(o, 16)]
            v = (plsc.load_gather(sq_v, [r16])
                 * plsc.load_gather(sq_v, [c16]))
            inv_v[j, pl.ds(o, 16)] = v / e16

    pltpu.sync_copy(inv_v, inv_hbm.at[w])


def _invn(sq, rowp, colp, ewp):
    kern = pl.kernel(
        _invn_body,
        out_type=jax.ShapeDtypeStruct((NW, E_CHUNKS, EC), jnp.float32),
        mesh=_MESH,
        scratch_types=[
            pltpu.VMEM((NPAD,), jnp.float32),
            pltpu.VMEM((E_CHUNKS, EC), jnp.int32),
            pltpu.VMEM((E_CHUNKS, EC), jnp.int32),
            pltpu.VMEM((E_CHUNKS, EC), jnp.float32),
            pltpu.VMEM((E_CHUNKS, EC), jnp.float32),
        ],
        compiler_params=_CP,
    )
    return kern(sq, rowp, colp, ewp)


def _edge_body(u_hbm, row_hbm, col_hbm, inv_hbm, z128_hbm, spart_hbm,
               rbuf, cbuf, ibuf, gbuf, sbuf, acc, gsems, ssems):
    core = lax.axis_index("c")
    sub = lax.axis_index("s")
    w = core * 16 + sub

    pltpu.sync_copy(_sub_rows(z128_hbm, sub), _sub_rows(acc, sub))
    plsc.subcore_barrier()

    def stage(g, p):
        pltpu.sync_copy(row_hbm.at[w].at[pl.ds(g * EG, EG)], rbuf.at[p])
        pltpu.sync_copy(col_hbm.at[w].at[pl.ds(g * EG, EG)], cbuf.at[p])
        pltpu.sync_copy(inv_hbm.at[w].at[pl.ds(g * EG, EG)], ibuf.at[p])

    def g_start(p, cc, b):
        pltpu.make_async_copy(u_hbm.at[rbuf.at[p].at[cc]], gbuf.at[b],
                              gsems.at[b]).start()

    def g_wait(p, cc, b):
        pltpu.make_async_copy(u_hbm.at[rbuf.at[p].at[cc]], gbuf.at[b],
                              gsems.at[b]).wait()

    def s_wait(p, cc, b):
        pltpu.make_async_copy(sbuf.at[b], acc.at[cbuf.at[p].at[cc]],
                              ssems.at[b]).wait()

    stage(0, 0)
    g_start(0, 0, 0)
    g_start(0, 1, 1)

    @pl.loop(0, NGROUPS, step=2)
    def _(gg):
        for p in range(2):
            g = gg + p
            for cc in range(4):
                j4 = g * 4 + cc
                b = cc % 2
                g_wait(p, cc, b)

                @pl.when(j4 >= 2)
                def _():
                    if cc < 2:
                        s_wait(1 - p, cc + 2, b)
                    else:
                        s_wait(p, cc - 2, b)

                gb = gbuf.at[b]
                sb = sbuf.at[b]
                iv = ibuf.at[p].at[cc]

                @plsc.parallel_loop(0, EC, unroll=2)
                def _(e):
                    inv16 = plsc.load_gather(
                        iv, [jnp.full((16,), e, jnp.int32)])
                    for t in range(8):
                        sl = pl.ds(t * 16, 16)
                        v = gb[e, sl] * inv16
                        sb[e, sl] = jnp.minimum(jnp.maximum(v, LO), HI)

                pltpu.async_copy(sbuf.at[b], acc.at[cbuf.at[p].at[cc]],
                                 ssems.at[b], add=True)

                if cc < 2:
                    g_start(p, cc + 2, b)
                    if cc == 1:
                        @pl.when(g + 1 < NGROUPS)
                        def _():
                            stage(g + 1, 1 - p)
                else:
                    @pl.when(g + 1 < NGROUPS)
                    def _():
                        g_start(1 - p, cc - 2, b)

    s_wait(1, 2, 0)
    s_wait(1, 3, 1)

    plsc.subcore_barrier()
    pltpu.sync_copy(_sub_rows(acc, sub),
                    spart_hbm.at[core].at[pl.ds(sub * RPS, RPS)])


def _edge_pass(u, rowp, colp, invp, z128):
    kern = pl.kernel(
        _edge_body,
        out_type=jax.ShapeDtypeStruct((2, NPAD, H), jnp.float32),
        mesh=_MESH,
        scratch_types=[
            pltpu.VMEM((2, EG, EC), jnp.int32),
            pltpu.VMEM((2, EG, EC), jnp.int32),
            pltpu.VMEM((2, EG, EC), jnp.float32),
            pltpu.VMEM((2, EC, H), jnp.float32),
            pltpu.VMEM((2, EC, H), jnp.float32),
            pltpu.VMEM_SHARED((NPAD, H), jnp.float32),
            pltpu.SemaphoreType.DMA((2,)),
            pltpu.SemaphoreType.DMA((2,)),
        ],
        compiler_params=_CP,
    )
    return kern(u, rowp, colp, invp, z128)


def _sq_body(dpart, sq_out, cnt_out):
    deg = dpart[0, :, 0:1] + dpart[1, :, 0:1] + 1.0
    cnt = dpart[0, :N, 1:2] + dpart[1, :N, 1:2] + 1.0
    sq_out[...] = jnp.sqrt(deg)
    cnt_out[...] = cnt


def _mm1_body(hpart, w1, u_out):
    h = hpart[0, :N, :] + hpart[1, :N, :]
    hw = jnp.dot(h, w1[...], preferred_element_type=jnp.float32)
    u_out[...] = jnp.where(hw > 0, 1.0 / hw, jnp.inf)


def _mm2_body(spart, cnt, b1, w2, u_out):
    s = spart[0, :N, :] + spart[1, :N, :]
    pre = jnp.clip(cnt[...] / s, LO, HI) + b1[...][None, :]
    h1 = jnp.maximum(pre, 0.0)
    hw = jnp.dot(h1, w2[...], preferred_element_type=jnp.float32)
    u_out[...] = jnp.where(hw > 0, 1.0 / hw, jnp.inf)


def _final_body(spart, cnt, b2, batch, lin_w, lin_b, out):
    s = spart[0, :N, :] + spart[1, :N, :]
    h2 = jnp.clip(cnt[...] / s, LO, HI) + b2[...][None, :]
    iota = lax.broadcasted_iota(jnp.int32, (N, NUM_GRAPHS), 1)
    oh = (batch[...] == iota).astype(jnp.float32)
    gsum = lax.dot_general(oh, h2, (((0,), (0,)), ((), ())),
                           preferred_element_type=jnp.float32)
    gcnt = jnp.sum(oh, axis=0)
    g = gsum / jnp.maximum(gcnt, 1.0)[:, None]
    out[...] = jnp.dot(g, lin_w[...],
                       preferred_element_type=jnp.float32) + lin_b[...][None, :]


def _tc_call(body, out_shape, *args):
    return pl.pallas_call(body, out_shape=out_shape)(*args)


@jax.jit
def kernel(x, edge_index, batch, edge_weight, atom_emb, W1, b1, W2, b2,
           lin_W, lin_b):
    f32, i32 = jnp.float32, jnp.int32
    row, col = edge_index[0], edge_index[1]

    emb_flat = atom_emb.reshape(NUM_ATOM_FEATS * ATOM_VOCAB, H)
    offs = (jnp.arange(NUM_ATOM_FEATS, dtype=i32) * ATOM_VOCAB)[None, :]
    aflat = (x + offs).reshape(-1)
    aflat = jnp.concatenate(
        [aflat, jnp.zeros((A_TOTAL - N * NUM_ATOM_FEATS,), i32)])
    aidx = aflat.reshape(NW, A_CHUNKS, C)
    adst = jnp.repeat(jnp.arange(N, dtype=i32), NUM_ATOM_FEATS)
    apad = A_TOTAL - N * NUM_ATOM_FEATS
    adst = jnp.concatenate(
        [adst, DUMMY + (jnp.arange(apad, dtype=i32) % (NPAD - N))])
    adst = adst.reshape(NW, A_CHUNKS, C)

    val16 = jnp.concatenate(
        [edge_weight[:, None], jnp.ones((E, 1), f32), jnp.zeros((E, 14), f32)],
        axis=1)
    val16 = jnp.concatenate([val16, jnp.zeros((D_TOTAL - E, 16), f32)])
    val16 = val16.reshape(NW, D_CHUNKS, C, 16)
    cidx_d = jnp.concatenate(
        [col, DUMMY + (jnp.arange(D_TOTAL - E, dtype=i32) % (NPAD - N))])
    cidx_d = cidx_d.reshape(NW, D_CHUNKS, C)

    loop = jnp.arange(N, dtype=i32)
    npadE = E_TOTAL - E2
    pad_col = DUMMY + (jnp.arange(npadE, dtype=i32) % (NPAD - N))
    rowp = jnp.concatenate([row, loop, jnp.zeros((npadE,), i32)])
    colp = jnp.concatenate([col, loop, pad_col])
    ewp = jnp.concatenate([edge_weight, jnp.ones((N + npadE,), f32)])
    rowp = rowp.reshape(NW, E_CHUNKS, EC)
    colp = colp.reshape(NW, E_CHUNKS, EC)
    ewp = ewp.reshape(NW, E_CHUNKS, EC)

    z128 = jnp.zeros((NPAD, H), f32)
    z16 = jnp.zeros((NPAD, 16), f32)

    hpart = _atom(emb_flat, aidx, adst, z128)
    dpart = _deg(val16, cidx_d, z16)

    sq, cnt = _tc_call(
        _sq_body,
        [jax.ShapeDtypeStruct((NPAD, 1), f32),
         jax.ShapeDtypeStruct((N, 1), f32)],
        dpart)
    sq1d = sq.reshape(NPAD)

    invp = _invn(sq1d, rowp, colp, ewp)
    u1 = _tc_call(_mm1_body, jax.ShapeDtypeStruct((N, H), f32), hpart, W1)
    spart1 = _edge_pass(u1, rowp, colp, invp, z128)

    u2 = _tc_call(_mm2_body, jax.ShapeDtypeStruct((N, H), f32),
                  spart1, cnt, b1, W2)

    spart2 = _edge_pass(u2, rowp, colp, invp, z128)

    out = _tc_call(_final_body,
                   jax.ShapeDtypeStruct((NUM_GRAPHS, NUM_TASKS), f32),
                   spart2, cnt, b2, batch[:, None], lin_W, lin_b)
    return out


# speedup vs baseline: 1.2744x; 1.2517x over previous
import dataclasses

import jax
import jax.numpy as jnp
from jax import lax
from jax.experimental import pallas as pl
from jax.experimental.pallas import tpu as pltpu
from jax.experimental.pallas import tpu_sc as plsc

N = 10000
E = 320000
H = 128
NUM_TASKS = 128
NUM_GRAPHS = 256
NUM_ATOM_FEATS = 9
ATOM_VOCAB = 119

NPAD = 10112
DUMMY = N
RPS = NPAD // 16

NW = 32
C = 128

A_CHUNKS = 22
A_TOTAL = NW * A_CHUNKS * C

D_CHUNKS = 80
D_TOTAL = NW * D_CHUNKS * C

EC = 64
EG = 4
NGROUPS = 42
E_CHUNKS = NGROUPS * EG
E_TOTAL = NW * E_CHUNKS * EC
E2 = E + N

_MESH = plsc.VectorSubcoreMesh(core_axis_name="c", subcore_axis_name="s")

_CP = pltpu.CompilerParams()
if "needs_layout_passes" in pltpu.CompilerParams.__dataclass_fields__:
    _CP = dataclasses.replace(_CP, needs_layout_passes=False)

LO = 0.01
HI = 1e7


def _sub_rows(ref, sub):
    return ref.at[pl.ds(sub * RPS, RPS)]


def _atom_body(emb_hbm, aidx_hbm, adst_hbm, z128_hbm, hpart_hbm,
               aidx_v, adst_v, gbuf, acc_h, gsems, ssems):
    core = lax.axis_index("c")
    sub = lax.axis_index("s")
    w = core * 16 + sub

    pltpu.sync_copy(_sub_rows(z128_hbm, sub), _sub_rows(acc_h, sub))
    plsc.subcore_barrier()

    pltpu.sync_copy(aidx_hbm.at[w], aidx_v)
    pltpu.sync_copy(adst_hbm.at[w], adst_v)

    def a_gather(j, b):
        return pltpu.make_async_copy(emb_hbm.at[aidx_v.at[j]], gbuf.at[b],
                                     gsems.at[b])

    def a_swait(j, b):
        pltpu.make_async_copy(gbuf.at[b], acc_h.at[adst_v.at[j]],
                              ssems.at[b]).wait()

    a_gather(0, 0).start()

    @pl.loop(0, A_CHUNKS, step=2)
    def _(k):
        for b in range(2):
            j = k + b
            o = 1 - b
            a_gather(j, b).wait()
            pltpu.async_copy(gbuf.at[b], acc_h.at[adst_v.at[j]],
                             ssems.at[b], add=True)

            @pl.when(j + 1 < A_CHUNKS)
            def _():
                @pl.when(j >= 1)
                def _():
                    a_swait(j - 1, o)
                a_gather(j + 1, o).start()

    a_swait(A_CHUNKS - 2, (A_CHUNKS - 2) % 2)
    a_swait(A_CHUNKS - 1, (A_CHUNKS - 1) % 2)

    plsc.subcore_barrier()
    pltpu.sync_copy(_sub_rows(acc_h, sub),
                    hpart_hbm.at[core].at[pl.ds(sub * RPS, RPS)])


def _atom(emb_flat, aidx, adst, z128):
    kern = pl.kernel(
        _atom_body,
        out_type=jax.ShapeDtypeStruct((2, NPAD, H), jnp.float32),
        mesh=_MESH,
        scratch_types=[
            pltpu.VMEM((A_CHUNKS, C), jnp.int32),
            pltpu.VMEM((A_CHUNKS, C), jnp.int32),
            pltpu.VMEM((2, C, H), jnp.float32),
            pltpu.VMEM_SHARED((NPAD, H), jnp.float32),
            pltpu.SemaphoreType.DMA((2,)),
            pltpu.SemaphoreType.DMA((2,)),
        ],
    )
    return kern(emb_flat, aidx, adst, z128)


def _deg_body(val_hbm, cidx_hbm, z16_hbm, dpart_hbm,
              cidx_v, vbuf, acc_d, gsems, ssems):
    core = lax.axis_index("c")
    sub = lax.axis_index("s")
    w = core * 16 + sub

    pltpu.sync_copy(_sub_rows(z16_hbm, sub), _sub_rows(acc_d, sub))
    plsc.subcore_barrier()

    pltpu.sync_copy(cidx_hbm.at[w], cidx_v)

    def d_load(j, b):
        return pltpu.make_async_copy(val_hbm.at[w].at[j], vbuf.at[b],
                                     gsems.at[b])

    def d_swait(j, b):
        pltpu.make_async_copy(vbuf.at[b], acc_d.at[cidx_v.at[j]],
                              ssems.at[b]).wait()

    d_load(0, 0).start()

    @pl.loop(0, D_CHUNKS, step=2)
    def _(k):
        for b in range(2):
            j = k + b
            o = 1 - b
            d_load(j, b).wait()
            pltpu.async_copy(vbuf.at[b], acc_d.at[cidx_v.at[j]],
                             ssems.at[b], add=True)

            @pl.when(j + 1 < D_CHUNKS)
            def _():
                @pl.when(j >= 1)
                def _():
                    d_swait(j - 1, o)
                d_load(j + 1, o).start()

    d_swait(D_CHUNKS - 2, (D_CHUNKS - 2) % 2)
    d_swait(D_CHUNKS - 1, (D_CHUNKS - 1) % 2)

    plsc.subcore_barrier()
    pltpu.sync_copy(_sub_rows(acc_d, sub),
                    dpart_hbm.at[core].at[pl.ds(sub * RPS, RPS)])


def _deg(val16, cidx_d, z16):
    kern = pl.kernel(
        _deg_body,
        out_type=jax.ShapeDtypeStruct((2, NPAD, 16), jnp.float32),
        mesh=_MESH,
        scratch_types=[
            pltpu.VMEM((D_CHUNKS, C), jnp.int32),
            pltpu.VMEM((2, C, 16), jnp.float32),
            pltpu.VMEM_SHARED((NPAD, 16), jnp.float32),
            pltpu.SemaphoreType.DMA((2,)),
            pltpu.SemaphoreType.DMA((2,)),
        ],
    )
    return kern(val16, cidx_d, z16)


def _invn_body(sq_hbm, row_hbm, col_hbm, ew_hbm, inv_hbm,
               sq_v, row_v, col_v, ew_v, inv_v):
    core = lax.axis_index("c")
    sub = lax.axis_index("s")
    w = core * 16 + sub

    pltpu.sync_copy(sq_hbm, sq_v)
    pltpu.sync_copy(row_hbm.at[w], row_v)
    pltpu.sync_copy(col_hbm.at[w], col_v)
    pltpu.sync_copy(ew_hbm.at[w], ew_v)

    @plsc.parallel_loop(0, E_CHUNKS, unroll=2)
    def _(j):
        for t in range(EC // 16):
            o = t * 16
            r16 = row_v[j, pl.ds(o, 16)]
            c16 = col_v[j, pl.ds(o, 16)]
            e16 = ew_v[j, pl.ds(o, 16)]
            v = (plsc.load_gather(sq_v, [r16])
                 * plsc.load_gather(sq_v, [c16]))
            inv_v[j, pl.ds(o, 16)] = v / e16

    pltpu.sync_copy(inv_v, inv_hbm.at[w])


def _invn(sq, rowp, colp, ewp):
    kern = pl.kernel(
        _invn_body,
        out_type=jax.ShapeDtypeStruct((NW, E_CHUNKS, EC), jnp.float32),
        mesh=_MESH,
        scratch_types=[
            pltpu.VMEM((NPAD,), jnp.float32),
            pltpu.VMEM((E_CHUNKS, EC), jnp.int32),
            pltpu.VMEM((E_CHUNKS, EC), jnp.int32),
            pltpu.VMEM((E_CHUNKS, EC), jnp.float32),
            pltpu.VMEM((E_CHUNKS, EC), jnp.float32),
        ],
        compiler_params=_CP,
    )
    return kern(sq, rowp, colp, ewp)


def _edge_body(u_hbm, row_hbm, col_hbm, inv_hbm, z128_hbm, spart_hbm,
               rbuf, cbuf, ibuf, gbuf, sbuf, acc, gsems, ssems):
    core = lax.axis_index("c")
    sub = lax.axis_index("s")
    w = core * 16 + sub

    pltpu.sync_copy(_sub_rows(z128_hbm, sub), _sub_rows(acc, sub))
    plsc.subcore_barrier()

    def stage(g, p):
        pltpu.sync_copy(row_hbm.at[w].at[pl.ds(g * EG, EG)], rbuf.at[p])
        pltpu.sync_copy(col_hbm.at[w].at[pl.ds(g * EG, EG)], cbuf.at[p])
        pltpu.sync_copy(inv_hbm.at[w].at[pl.ds(g * EG, EG)], ibuf.at[p])

    def g_start(p, cc, b):
        pltpu.make_async_copy(u_hbm.at[rbuf.at[p].at[cc]], gbuf.at[b],
                              gsems.at[b]).start()

    def g_wait(p, cc, b):
        pltpu.make_async_copy(u_hbm.at[rbuf.at[p].at[cc]], gbuf.at[b],
                              gsems.at[b]).wait()

    def s_wait(p, cc, b):
        pltpu.make_async_copy(sbuf.at[b], acc.at[cbuf.at[p].at[cc]],
                              ssems.at[b]).wait()

    stage(0, 0)
    g_start(0, 0, 0)
    g_start(0, 1, 1)

    @pl.loop(0, NGROUPS, step=2)
    def _(gg):
        for p in range(2):
            g = gg + p
            for cc in range(4):
                j4 = g * 4 + cc
                b = cc % 2
                g_wait(p, cc, b)

                @pl.when(j4 >= 2)
                def _():
                    if cc < 2:
                        s_wait(1 - p, cc + 2, b)
                    else:
                        s_wait(p, cc - 2, b)

                gb = gbuf.at[b]
                sb = sbuf.at[b]
                iv = ibuf.at[p].at[cc]

                @plsc.parallel_loop(0, EC, unroll=2)
                def _(e):
                    inv16 = plsc.load_gather(
                        iv, [jnp.full((16,), e, jnp.int32)])
                    for t in range(8):
                        sl = pl.ds(t * 16, 16)
                        v = gb[e, sl] * inv16
                        sb[e, sl] = jnp.minimum(jnp.maximum(v, LO), HI)

                pltpu.async_copy(sbuf.at[b], acc.at[cbuf.at[p].at[cc]],
                                 ssems.at[b], add=True)

                if cc < 2:
                    g_start(p, cc + 2, b)
                    if cc == 1:
                        @pl.when(g + 1 < NGROUPS)
                        def _():
                            stage(g + 1, 1 - p)
                else:
                    @pl.when(g + 1 < NGROUPS)
                    def _():
                        g_start(1 - p, cc - 2, b)

    s_wait(1, 2, 0)
    s_wait(1, 3, 1)

    plsc.subcore_barrier()
    pltpu.sync_copy(_sub_rows(acc, sub),
                    spart_hbm.at[core].at[pl.ds(sub * RPS, RPS)])


def _edge_pass(u, rowp, colp, invp, z128):
    kern = pl.kernel(
        _edge_body,
        out_type=jax.ShapeDtypeStruct((2, NPAD, H), jnp.float32),
        mesh=_MESH,
        scratch_types=[
            pltpu.VMEM((2, EG, EC), jnp.int32),
            pltpu.VMEM((2, EG, EC), jnp.int32),
            pltpu.VMEM((2, EG, EC), jnp.float32),
            pltpu.VMEM((2, EC, H), jnp.float32),
            pltpu.VMEM((2, EC, H), jnp.float32),
            pltpu.VMEM_SHARED((NPAD, H), jnp.float32),
            pltpu.SemaphoreType.DMA((2,)),
            pltpu.SemaphoreType.DMA((2,)),
        ],
        compiler_params=_CP,
    )
    return kern(u, rowp, colp, invp, z128)


def _sq_body(dpart, sq_out, cnt_out):
    deg = dpart[0, :, 0:1] + dpart[1, :, 0:1] + 1.0
    cnt = dpart[0, :N, 1:2] + dpart[1, :N, 1:2] + 1.0
    sq_out[...] = jnp.sqrt(deg)
    cnt_out[...] = cnt


def _mm1_body(hpart, w1, u_out):
    h = hpart[0, :N, :] + hpart[1, :N, :]
    hw = jnp.dot(h, w1[...], preferred_element_type=jnp.float32)
    u_out[...] = jnp.where(hw > 0, 1.0 / hw, jnp.inf)


def _mm2_body(spart, cnt, b1, w2, u_out):
    s = spart[0, :N, :] + spart[1, :N, :]
    pre = jnp.clip(cnt[...] / s, LO, HI) + b1[...][None, :]
    h1 = jnp.maximum(pre, 0.0)
    hw = jnp.dot(h1, w2[...], preferred_element_type=jnp.float32)
    u_out[...] = jnp.where(hw > 0, 1.0 / hw, jnp.inf)


def _final_body(spart, cnt, b2, batch, lin_w, lin_b, out):
    s = spart[0, :N, :] + spart[1, :N, :]
    h2 = jnp.clip(cnt[...] / s, LO, HI) + b2[...][None, :]
    iota = lax.broadcasted_iota(jnp.int32, (N, NUM_GRAPHS), 1)
    oh = (batch[...] == iota).astype(jnp.float32)
    gsum = lax.dot_general(oh, h2, (((0,), (0,)), ((), ())),
                           preferred_element_type=jnp.float32)
    gcnt = jnp.sum(oh, axis=0)
    g = gsum / jnp.maximum(gcnt, 1.0)[:, None]
    out[...] = jnp.dot(g, lin_w[...],
                       preferred_element_type=jnp.float32) + lin_b[...][None, :]


def _tc_call(body, out_shape, *args):
    return pl.pallas_call(body, out_shape=out_shape)(*args)


@jax.jit
def kernel(x, edge_index, batch, edge_weight, atom_emb, W1, b1, W2, b2,
           lin_W, lin_b):
    f32, i32 = jnp.float32, jnp.int32
    row, col = edge_index[0], edge_index[1]

    emb_flat = atom_emb.reshape(NUM_ATOM_FEATS * ATOM_VOCAB, H)
    offs = (jnp.arange(NUM_ATOM_FEATS, dtype=i32) * ATOM_VOCAB)[None, :]
    aflat = (x + offs).reshape(-1)
    aflat = jnp.concatenate(
        [aflat, jnp.zeros((A_TOTAL - N * NUM_ATOM_FEATS,), i32)])
    aidx = aflat.reshape(NW, A_CHUNKS, C)
    adst = jnp.repeat(jnp.arange(N, dtype=i32), NUM_ATOM_FEATS)
    apad = A_TOTAL - N * NUM_ATOM_FEATS
    adst = jnp.concatenate(
        [adst, DUMMY + (jnp.arange(apad, dtype=i32) % (NPAD - N))])
    adst = adst.reshape(NW, A_CHUNKS, C)

    val16 = jnp.concatenate(
        [edge_weight[:, None], jnp.ones((E, 1), f32), jnp.zeros((E, 14), f32)],
        axis=1)
    val16 = jnp.concatenate([val16, jnp.zeros((D_TOTAL - E, 16), f32)])
    val16 = val16.reshape(NW, D_CHUNKS, C, 16)
    cidx_d = jnp.concatenate(
        [col, DUMMY + (jnp.arange(D_TOTAL - E, dtype=i32) % (NPAD - N))])
    cidx_d = cidx_d.reshape(NW, D_CHUNKS, C)

    loop = jnp.arange(N, dtype=i32)
    npadE = E_TOTAL - E2
    pad_col = DUMMY + (jnp.arange(npadE, dtype=i32) % (NPAD - N))
    rowp = jnp.concatenate([row, loop, jnp.zeros((npadE,), i32)])
    colp = jnp.concatenate([col, loop, pad_col])
    ewp = jnp.concatenate([edge_weight, jnp.ones((N + npadE,), f32)])
    def _bal(a):
        return a.reshape(E_CHUNKS, NW, EC).transpose(1, 0, 2)

    rowp = _bal(rowp)
    colp = _bal(colp)
    ewp = _bal(ewp)

    z128 = jnp.zeros((NPAD, H), f32)
    z16 = jnp.zeros((NPAD, 16), f32)

    hpart = _atom(emb_flat, aidx, adst, z128)
    dpart = _deg(val16, cidx_d, z16)

    sq, cnt = _tc_call(
        _sq_body,
        [jax.ShapeDtypeStruct((NPAD, 1), f32),
         jax.ShapeDtypeStruct((N, 1), f32)],
        dpart)
    sq1d = sq.reshape(NPAD)

    invp = _invn(sq1d, rowp, colp, ewp)
    u1 = _tc_call(_mm1_body, jax.ShapeDtypeStruct((N, H), f32), hpart, W1)
    spart1 = _edge_pass(u1, rowp, colp, invp, z128)

    u2 = _tc_call(_mm2_body, jax.ShapeDtypeStruct((N, H), f32),
                  spart1, cnt, b1, W2)

    spart2 = _edge_pass(u2, rowp, colp, invp, z128)

    out = _tc_call(_final_body,
                   jax.ShapeDtypeStruct((NUM_GRAPHS, NUM_TASKS), f32),
                   spart2, cnt, b2, batch[:, None], lin_W, lin_b)
    return out
